# baseline (speedup 1.0000x reference)
import functools

import jax
import jax.numpy as jnp
from jax import lax
from jax.experimental import pallas as pl
from jax.experimental.pallas import tpu as pltpu
from jax.experimental.pallas import tpu_sc as plsc

B = 16384
K = 16
NC = 2
NS = 16
NW = NC * NS
BPW = B // NW
CHUNK = 128
NCHUNK = BPW // CHUNK


def _sc_body(uid_hbm, iid_hbm, uw_hbm, iw_hbm, out_hbm,
             uidx_v, iidx_v, urows, irows, outv, sem_u, sem_i):
    wid = lax.axis_index("s") * NC + lax.axis_index("c")
    row0 = wid * NCHUNK

    pltpu.sync_copy(uid_hbm.at[pl.ds(row0, NCHUNK), :], uidx_v)
    pltpu.sync_copy(iid_hbm.at[pl.ds(row0, NCHUNK), :], iidx_v)

    copies = []
    for j in range(NCHUNK):
        copies.append(pltpu.async_copy(
            uw_hbm.at[uidx_v.at[j]], urows.at[pl.ds(j * CHUNK, CHUNK), :],
            sem_u))
        copies.append(pltpu.async_copy(
            iw_hbm.at[iidx_v.at[j]], irows.at[pl.ds(j * CHUNK, CHUNK), :],
            sem_i))
    for c in copies:
        c.wait()

    iota = lax.iota(jnp.int32, K)

    def group(g, carry):
        rows = g * K + iota
        acc = jnp.zeros((K,), jnp.float32)
        for k in range(K):
            col = jnp.full((K,), k, jnp.int32)
            uc = plsc.load_gather(urows, [rows, col])
            ic = plsc.load_gather(irows, [rows, col])
            acc = acc + uc * ic
        outv[pl.ds(g * K, K)] = acc
        return carry

    lax.fori_loop(0, BPW // K, group, 0)

    pltpu.sync_copy(outv, out_hbm.at[pl.ds(wid * BPW, BPW)])


@jax.jit
def kernel(train_x, user_weight, item_weight):
    uid = train_x[:, 0].reshape(B // CHUNK, CHUNK)
    iid = train_x[:, 1].reshape(B // CHUNK, CHUNK)

    mesh = plsc.VectorSubcoreMesh(
        core_axis_name="c", subcore_axis_name="s",
        num_cores=NC, num_subcores=NS)
    fn = pl.kernel(
        _sc_body,
        out_type=jax.ShapeDtypeStruct((B,), jnp.float32),
        mesh=mesh,
        scratch_types=[
            pltpu.VMEM((NCHUNK, CHUNK), jnp.int32),
            pltpu.VMEM((NCHUNK, CHUNK), jnp.int32),
            pltpu.VMEM((BPW, K), jnp.float32),
            pltpu.VMEM((BPW, K), jnp.float32),
            pltpu.VMEM((BPW,), jnp.float32),
            pltpu.SemaphoreType.DMA,
            pltpu.SemaphoreType.DMA,
        ],
        compiler_params=pltpu.CompilerParams(
            needs_layout_passes=False, use_tc_tiling_on_sc=False),
    )
    return fn(uid, iid, user_weight, item_weight)
